# baseline (device time: 18232 ns/iter reference)
import jax
import jax.numpy as jnp
from jax import lax
from jax.experimental import pallas as pl
from jax.experimental.pallas import tpu as pltpu

_SPLIT = 8


def kernel(x, dy, gamma):
    del gamma
    m, d = x.shape
    rows = m // _SPLIT

    def body(
        x_hbm, dy_hbm, out_ref,
        xb, dyb, partial_ref, recv_ref,
        copy_sems, send_sems, recv_sems,
    ):
        mx = lax.axis_index("x")
        my = lax.axis_index("y")
        mz = lax.axis_index("z")
        off = (mx * 4 + mz) * rows

        cp_x = pltpu.make_async_copy(
            x_hbm.at[pl.ds(off, rows), :], xb, copy_sems.at[0]
        )
        cp_dy = pltpu.make_async_copy(
            dy_hbm.at[pl.ds(off, rows), :], dyb, copy_sems.at[1]
        )
        cp_x.start()
        cp_dy.start()

        peers = [
            (1 - mx, my, mz),
            (mx, my, mz ^ 1),
            (mx, my, mz ^ 2),
        ]

        barrier_sem = pltpu.get_barrier_semaphore()
        for peer in peers:
            pl.semaphore_signal(
                barrier_sem, inc=1, device_id=peer,
                device_id_type=pl.DeviceIdType.MESH,
            )
        pl.semaphore_wait(barrier_sem, len(peers))

        cp_x.wait()
        cp_dy.wait()

        xv = xb[:, :]
        dyv = dyb[:, :]
        mu = jnp.mean(xv, axis=1, keepdims=True)
        var = jnp.mean((xv - mu) * (xv - mu), axis=1, keepdims=True)
        rstd = lax.rsqrt(var + 1e-5)
        xhat = (xv - mu) * rstd
        partial_ref[0, :] = jnp.sum(dyv * xhat, axis=0)
        partial_ref[1, :] = jnp.sum(dyv, axis=0)

        for i, peer in enumerate(peers):
            rdma = pltpu.make_async_remote_copy(
                src_ref=partial_ref,
                dst_ref=recv_ref.at[i],
                send_sem=send_sems.at[i],
                recv_sem=recv_sems.at[i],
                device_id=peer,
                device_id_type=pl.DeviceIdType.MESH,
            )
            rdma.start()
            rdma.wait()
            partial_ref[:, :] = partial_ref[:, :] + recv_ref[i]

        out_ref[:, :] = partial_ref[:, :]

    return pl.pallas_call(
        body,
        out_shape=jax.ShapeDtypeStruct((2, d), jnp.float32),
        in_specs=[
            pl.BlockSpec(memory_space=pl.ANY),
            pl.BlockSpec(memory_space=pl.ANY),
        ],
        out_specs=pl.BlockSpec(memory_space=pltpu.VMEM),
        scratch_shapes=[
            pltpu.VMEM((rows, d), jnp.float32),
            pltpu.VMEM((rows, d), jnp.float32),
            pltpu.VMEM((2, d), jnp.float32),
            pltpu.VMEM((3, 2, d), jnp.float32),
            pltpu.SemaphoreType.DMA((2,)),
            pltpu.SemaphoreType.DMA((3,)),
            pltpu.SemaphoreType.DMA((3,)),
        ],
        compiler_params=pltpu.CompilerParams(collective_id=0),
    )(x, dy)


# device time: 16338 ns/iter; 1.1159x vs baseline; 1.1159x over previous
import jax
import jax.numpy as jnp
from jax import lax
from jax.experimental import pallas as pl
from jax.experimental.pallas import tpu as pltpu

_SPLIT = 8


def kernel(x, dy, gamma):
    del gamma
    m, d = x.shape
    rows = m // _SPLIT

    def body(
        x_hbm, dy_hbm, out_ref,
        xb, dyb, partial_ref, recv_ref,
        copy_sems, send_sems, recv_sems,
    ):
        mx = lax.axis_index("x")
        my = lax.axis_index("y")
        mz = lax.axis_index("z")
        off = (mx * 4 + mz) * rows

        cp_x = pltpu.make_async_copy(
            x_hbm.at[pl.ds(off, rows), :], xb, copy_sems.at[0]
        )
        cp_dy = pltpu.make_async_copy(
            dy_hbm.at[pl.ds(off, rows), :], dyb, copy_sems.at[1]
        )
        cp_x.start()
        cp_dy.start()

        peers = [
            (mx ^ dx, my, mz ^ dz)
            for dx in (0, 1)
            for dz in (0, 1, 2, 3)
            if (dx, dz) != (0, 0)
        ]

        barrier_sem = pltpu.get_barrier_semaphore()
        for peer in peers:
            pl.semaphore_signal(
                barrier_sem, inc=1, device_id=peer,
                device_id_type=pl.DeviceIdType.MESH,
            )
        pl.semaphore_wait(barrier_sem, len(peers))

        cp_x.wait()
        cp_dy.wait()

        xv = xb[:, :]
        dyv = dyb[:, :]
        mu = jnp.mean(xv, axis=1, keepdims=True)
        var = jnp.mean((xv - mu) * (xv - mu), axis=1, keepdims=True)
        rstd = lax.rsqrt(var + 1e-5)
        xhat = (xv - mu) * rstd
        partial_ref[0, :] = jnp.sum(dyv * xhat, axis=0)
        partial_ref[1, :] = jnp.sum(dyv, axis=0)

        rdmas = []
        for i, peer in enumerate(peers):
            rdma = pltpu.make_async_remote_copy(
                src_ref=partial_ref,
                dst_ref=recv_ref.at[i],
                send_sem=send_sems.at[i],
                recv_sem=recv_sems.at[i],
                device_id=peer,
                device_id_type=pl.DeviceIdType.MESH,
            )
            rdma.start()
            rdmas.append(rdma)

        acc = partial_ref[:, :]
        for i, rdma in enumerate(rdmas):
            rdma.wait_recv()
            acc = acc + recv_ref[i]
        out_ref[:, :] = acc
        for rdma in rdmas:
            rdma.wait_send()

    return pl.pallas_call(
        body,
        out_shape=jax.ShapeDtypeStruct((2, d), jnp.float32),
        in_specs=[
            pl.BlockSpec(memory_space=pl.ANY),
            pl.BlockSpec(memory_space=pl.ANY),
        ],
        out_specs=pl.BlockSpec(memory_space=pltpu.VMEM),
        scratch_shapes=[
            pltpu.VMEM((rows, d), jnp.float32),
            pltpu.VMEM((rows, d), jnp.float32),
            pltpu.VMEM((2, d), jnp.float32),
            pltpu.VMEM((7, 2, d), jnp.float32),
            pltpu.SemaphoreType.DMA((2,)),
            pltpu.SemaphoreType.DMA((7,)),
            pltpu.SemaphoreType.DMA((7,)),
        ],
        compiler_params=pltpu.CompilerParams(collective_id=0),
    )(x, dy)
